# baseline (speedup 1.0000x reference)
import functools

import jax
import jax.numpy as jnp
from jax import lax
from jax.experimental import pallas as pl
from jax.experimental.pallas import tpu as pltpu
from jax.experimental.pallas import tpu_sc as plsc

BATCH = 16384
EMBED_DIM = 64
TABLE_ROWS = 1000000
LANES = 16
NUM_CORES = 2
NUM_SUBCORES = 16
NUM_WORKERS = NUM_CORES * NUM_SUBCORES
BPW = BATCH // NUM_WORKERS
CHUNK = 128
NCHUNK = BPW // CHUNK


def _mf_body(ui_ref, ii_ref, ut_ref, it_ref, bsum_ref, out_ref,
             uidx_v, iidx_v, uibuf_v, iibuf_v, ucols_v, icols_v,
             bias_v, out_v, sem):
    wid = lax.axis_index("s") * NUM_CORES + lax.axis_index("c")
    base = wid * BPW

    pltpu.sync_copy(ui_ref.at[pl.ds(base, BPW)], uidx_v)
    pltpu.sync_copy(ii_ref.at[pl.ds(base, BPW)], iidx_v)
    pltpu.sync_copy(bsum_ref.at[pl.ds(base, BPW)], bias_v)

    for c in range(NCHUNK):
        c0 = c * CHUNK

        def gen(i, carry):
            sl = pl.ds(i * LANES, LANES)
            ubase = uidx_v[pl.ds(c0 + i * LANES, LANES)]
            ibase = iidx_v[pl.ds(c0 + i * LANES, LANES)]
            for d in range(EMBED_DIM):
                uibuf_v[d, sl] = ubase + d * TABLE_ROWS
                iibuf_v[d, sl] = ibase + d * TABLE_ROWS
            return carry

        lax.fori_loop(0, CHUNK // LANES, gen, 0)

        copies = []
        for d in range(EMBED_DIM):
            copies.append(
                pltpu.async_copy(ut_ref.at[uibuf_v.at[d]], ucols_v.at[d], sem))
            copies.append(
                pltpu.async_copy(it_ref.at[iibuf_v.at[d]], icols_v.at[d], sem))
        for cp in copies:
            cp.wait()

        def group(g, carry):
            sl = pl.ds(g * LANES, LANES)
            acc = bias_v[pl.ds(c0 + g * LANES, LANES)]
            for d in range(EMBED_DIM):
                acc = acc + ucols_v[d, sl] * icols_v[d, sl]
            out_v[pl.ds(c0 + g * LANES, LANES)] = acc
            return carry

        lax.fori_loop(0, CHUNK // LANES, group, 0)

    pltpu.sync_copy(out_v, out_ref.at[pl.ds(base, BPW)])


@functools.partial(
    pl.kernel,
    out_type=jax.ShapeDtypeStruct((BATCH,), jnp.float32),
    mesh=plsc.VectorSubcoreMesh(core_axis_name="c", subcore_axis_name="s"),
    compiler_params=pltpu.CompilerParams(use_tc_tiling_on_sc=False),
    scratch_types=[
        pltpu.VMEM((BPW,), jnp.int32),
        pltpu.VMEM((BPW,), jnp.int32),
        pltpu.VMEM((EMBED_DIM, CHUNK), jnp.int32),
        pltpu.VMEM((EMBED_DIM, CHUNK), jnp.int32),
        pltpu.VMEM((EMBED_DIM, CHUNK), jnp.float32),
        pltpu.VMEM((EMBED_DIM, CHUNK), jnp.float32),
        pltpu.VMEM((BPW,), jnp.float32),
        pltpu.VMEM((BPW,), jnp.float32),
        pltpu.SemaphoreType.DMA,
    ],
)
def _mf_kernel(ui, ii, ut, it, bsum, out,
               uidx_v, iidx_v, uibuf_v, iibuf_v, ucols_v, icols_v,
               bias_v, out_v, sem):
    _mf_body(ui, ii, ut, it, bsum, out,
             uidx_v, iidx_v, uibuf_v, iibuf_v, ucols_v, icols_v,
             bias_v, out_v, sem)


def kernel(user_indices, item_indices, user_emb, item_emb, user_bias, item_bias):
    ui = user_indices.astype(jnp.int32)
    ii = item_indices.astype(jnp.int32)
    ut = user_emb.T.reshape(-1)
    it = item_emb.T.reshape(-1)
    user_b = jnp.take(user_bias, user_indices, axis=0).squeeze(-1)
    item_b = jnp.take(item_bias, item_indices, axis=0).squeeze(-1)
    bsum = user_b + item_b
    return _mf_kernel(ui, ii, ut, it, bsum)


# speedup vs baseline: 9.2488x; 9.2488x over previous
import functools

import jax
import jax.numpy as jnp
from jax import lax
from jax.experimental import pallas as pl
from jax.experimental.pallas import tpu as pltpu
from jax.experimental.pallas import tpu_sc as plsc

BATCH = 16384
EMBED_DIM = 64
LANES = 16
NUM_CORES = 2
NUM_SUBCORES = 16
NUM_WORKERS = NUM_CORES * NUM_SUBCORES
BPW = BATCH // NUM_WORKERS
CHUNK = 128
NCHUNK = BPW // CHUNK


def _mf_body(ui_ref, ii_ref, ue_ref, ie_ref, ub_ref, ib_ref, out_ref,
             uidx_v, iidx_v, urows_v, irows_v, ubias_v, ibias_v, out_v, sem):
    wid = lax.axis_index("s") * NUM_CORES + lax.axis_index("c")
    base = wid * BPW

    pltpu.sync_copy(ui_ref.at[pl.ds(wid * NCHUNK, NCHUNK)], uidx_v)
    pltpu.sync_copy(ii_ref.at[pl.ds(wid * NCHUNK, NCHUNK)], iidx_v)

    copies = []
    for j in range(NCHUNK):
        sl = pl.ds(j * CHUNK, CHUNK)
        copies.append(pltpu.async_copy(ue_ref.at[uidx_v.at[j]], urows_v.at[sl], sem))
        copies.append(pltpu.async_copy(ie_ref.at[iidx_v.at[j]], irows_v.at[sl], sem))
        copies.append(pltpu.async_copy(ub_ref.at[uidx_v.at[j]], ubias_v.at[sl], sem))
        copies.append(pltpu.async_copy(ib_ref.at[iidx_v.at[j]], ibias_v.at[sl], sem))
    for c in copies:
        c.wait()

    lane_iota = lax.iota(jnp.int32, LANES)
    perms = [(lane_iota + s) & (LANES - 1) for s in (8, 4, 2, 1)]
    onehots = [
        jnp.where(lane_iota == k, jnp.float32(1.0), jnp.float32(0.0))
        for k in range(LANES)
    ]

    def group(g, carry):
        r0 = g * LANES
        acc = ubias_v[pl.ds(r0, LANES)] + ibias_v[pl.ds(r0, LANES)]
        for l in range(LANES):
            r = r0 + l
            p = (urows_v[r, pl.ds(0, LANES)] * irows_v[r, pl.ds(0, LANES)]
                 + urows_v[r, pl.ds(LANES, LANES)] * irows_v[r, pl.ds(LANES, LANES)]
                 + urows_v[r, pl.ds(2 * LANES, LANES)] * irows_v[r, pl.ds(2 * LANES, LANES)]
                 + urows_v[r, pl.ds(3 * LANES, LANES)] * irows_v[r, pl.ds(3 * LANES, LANES)])
            for perm in perms:
                p = p + p.at[perm].get(mode="promise_in_bounds", unique_indices=True)
            acc = acc + p * onehots[l]
        out_v[pl.ds(r0, LANES)] = acc
        return carry

    lax.fori_loop(0, BPW // LANES, group, 0)

    pltpu.sync_copy(out_v, out_ref.at[pl.ds(base, BPW)])


@functools.partial(
    pl.kernel,
    out_type=jax.ShapeDtypeStruct((BATCH,), jnp.float32),
    mesh=plsc.VectorSubcoreMesh(core_axis_name="c", subcore_axis_name="s"),
    compiler_params=pltpu.CompilerParams(use_tc_tiling_on_sc=False),
    scratch_types=[
        pltpu.VMEM((NCHUNK, CHUNK), jnp.int32),
        pltpu.VMEM((NCHUNK, CHUNK), jnp.int32),
        pltpu.VMEM((BPW, EMBED_DIM), jnp.float32),
        pltpu.VMEM((BPW, EMBED_DIM), jnp.float32),
        pltpu.VMEM((BPW,), jnp.float32),
        pltpu.VMEM((BPW,), jnp.float32),
        pltpu.VMEM((BPW,), jnp.float32),
        pltpu.SemaphoreType.DMA,
    ],
)
def _mf_kernel(ui, ii, ue, ie, ub, ib, out,
               uidx_v, iidx_v, urows_v, irows_v, ubias_v, ibias_v, out_v, sem):
    _mf_body(ui, ii, ue, ie, ub, ib, out,
             uidx_v, iidx_v, urows_v, irows_v, ubias_v, ibias_v, out_v, sem)


def kernel(user_indices, item_indices, user_emb, item_emb, user_bias, item_bias):
    ui = user_indices.astype(jnp.int32).reshape(NUM_WORKERS * NCHUNK, CHUNK)
    ii = item_indices.astype(jnp.int32).reshape(NUM_WORKERS * NCHUNK, CHUNK)
    ub = user_bias.reshape(-1)
    ib = item_bias.reshape(-1)
    return _mf_kernel(ui, ii, user_emb, item_emb, ub, ib)
